# baseline (device time: 23182 ns/iter reference)
import jax
import jax.numpy as jnp
from jax import lax
from jax.experimental import pallas as pl
from jax.experimental.pallas import tpu as pltpu

N_DEV = 8
B, SQ, SKV, HQ, DH = 2, 128, 128, 32, 64
H_LOC = HQ // N_DEV
DMODEL = 512
ROWS = B * SQ
CH = ROWS // N_DEV
CPB = SQ // CH


def kernel(x, Wq, K_ext, V_ext, Wo):
    def body(x_ref, wq_ref, k_hbm, v_hbm, wo_ref, out_ref,
             k_ref, v_ref, mine_ref, rs_ref, red_ref, ag_ref,
             kv_sems, rs_send, rs_recv, ag_send, ag_recv):
        me = lax.axis_index("i")
        h0 = me * H_LOC

        barrier_sem = pltpu.get_barrier_semaphore()
        for d in range(1, N_DEV):
            tgt = lax.rem(me + d, N_DEV)
            pl.semaphore_signal(
                barrier_sem, inc=1,
                device_id=(tgt,), device_id_type=pl.DeviceIdType.MESH,
            )

        k_cp = pltpu.make_async_copy(
            k_hbm.at[:, :, pl.ds(h0, H_LOC), :], k_ref, kv_sems.at[0])
        v_cp = pltpu.make_async_copy(
            v_hbm.at[:, :, pl.ds(h0, H_LOC), :], v_ref, kv_sems.at[1])
        k_cp.start()
        v_cp.start()

        x2d = x_ref[...].reshape(ROWS, DMODEL)
        q = jnp.dot(x2d, wq_ref[...], preferred_element_type=jnp.float32)

        qb = lax.broadcasted_iota(jnp.int32, (SQ, SKV), 0) // 64
        kb = lax.broadcasted_iota(jnp.int32, (SQ, SKV), 1) // 64
        mask = (qb == kb) | ((kb % 4) == (qb % 4))

        k_cp.wait()
        v_cp.wait()

        ctx_rows = []
        for b in range(B):
            ctx_cols = []
            for h in range(H_LOC):
                q_bh = q[b * SQ:(b + 1) * SQ, h * DH:(h + 1) * DH]
                k_bh = k_ref[b, :, h, :]
                v_bh = v_ref[b, :, h, :]
                s = lax.dot_general(
                    q_bh, k_bh, (((1,), (1,)), ((), ())),
                    preferred_element_type=jnp.float32,
                ) * 0.125
                s = jnp.where(mask, s, -1e9)
                s = s - jnp.max(s, axis=-1, keepdims=True)
                w = jnp.exp(s)
                w = w / jnp.sum(w, axis=-1, keepdims=True)
                ctx_cols.append(
                    jnp.dot(w, v_bh, preferred_element_type=jnp.float32))
            ctx_rows.append(jnp.concatenate(ctx_cols, axis=1))
        ctx = jnp.concatenate(ctx_rows, axis=0)
        partial = jnp.dot(ctx, wo_ref[...],
                          preferred_element_type=jnp.float32)
        mine_ref[...] = partial.astype(jnp.bfloat16)

        pl.semaphore_wait(barrier_sem, N_DEV - 1)
        tgts = [lax.rem(me + d, N_DEV) for d in range(1, N_DEV)]
        rs = []
        for d in range(1, N_DEV):
            rdma = pltpu.make_async_remote_copy(
                src_ref=mine_ref.at[pl.ds(tgts[d - 1] * CH, CH), :],
                dst_ref=rs_ref.at[d - 1],
                send_sem=rs_send.at[d - 1],
                recv_sem=rs_recv.at[d - 1],
                device_id=(tgts[d - 1],),
                device_id_type=pl.DeviceIdType.MESH,
            )
            rdma.start()
            rs.append(rdma)

        red = mine_ref[pl.ds(me * CH, CH), :].astype(jnp.float32)
        for d in range(1, N_DEV):
            rs[d - 1].wait_recv()
            red = red + rs_ref[d - 1].astype(jnp.float32)
        red_ref[...] = red.astype(jnp.bfloat16)

        ag = []
        for d in range(1, N_DEV):
            rdma = pltpu.make_async_remote_copy(
                src_ref=red_ref,
                dst_ref=ag_ref.at[d - 1],
                send_sem=ag_send.at[d - 1],
                recv_sem=ag_recv.at[d - 1],
                device_id=(tgts[d - 1],),
                device_id_type=pl.DeviceIdType.MESH,
            )
            rdma.start()
            ag.append(rdma)

        def out_store(c, val_f32):
            out_ref[pl.ds(c // CPB, 1), pl.ds(lax.rem(c, CPB) * CH, CH), :] = (
                val_f32.reshape(1, CH, DMODEL))

        out_store(me, red)
        for d in range(1, N_DEV):
            ag[d - 1].wait_recv()
            src = lax.rem(me - d + N_DEV, N_DEV)
            out_store(src, ag_ref[d - 1].astype(jnp.float32))

        for d in range(1, N_DEV):
            rs[d - 1].wait_send()
            ag[d - 1].wait_send()

    return pl.pallas_call(
        body,
        out_shape=jax.ShapeDtypeStruct((B, SQ, DMODEL), jnp.float32),
        in_specs=[
            pl.BlockSpec(memory_space=pltpu.VMEM),
            pl.BlockSpec(memory_space=pltpu.VMEM),
            pl.BlockSpec(memory_space=pltpu.MemorySpace.HBM),
            pl.BlockSpec(memory_space=pltpu.MemorySpace.HBM),
            pl.BlockSpec(memory_space=pltpu.VMEM),
        ],
        out_specs=pl.BlockSpec(memory_space=pltpu.VMEM),
        scratch_shapes=[
            pltpu.VMEM((B, SKV, H_LOC, DH), jnp.float32),
            pltpu.VMEM((B, SKV, H_LOC, DH), jnp.float32),
            pltpu.VMEM((ROWS, DMODEL), jnp.bfloat16),
            pltpu.VMEM((N_DEV - 1, CH, DMODEL), jnp.bfloat16),
            pltpu.VMEM((CH, DMODEL), jnp.bfloat16),
            pltpu.VMEM((N_DEV - 1, CH, DMODEL), jnp.bfloat16),
            pltpu.SemaphoreType.DMA((2,)),
            pltpu.SemaphoreType.DMA((N_DEV - 1,)),
            pltpu.SemaphoreType.DMA((N_DEV - 1,)),
            pltpu.SemaphoreType.DMA((N_DEV - 1,)),
            pltpu.SemaphoreType.DMA((N_DEV - 1,)),
        ],
        compiler_params=pltpu.CompilerParams(collective_id=0),
    )(x, Wq, K_ext, V_ext, Wo)


# device time: 14607 ns/iter; 1.5870x vs baseline; 1.5870x over previous
import jax
import jax.numpy as jnp
from jax import lax
from jax.experimental import pallas as pl
from jax.experimental.pallas import tpu as pltpu

N_DEV = 8
B, SQ, SKV, HQ, DH = 2, 128, 128, 32, 64
H_LOC = HQ // N_DEV
DMODEL = 512
ROWS = B * SQ
CH = ROWS // N_DEV
CPB = SQ // CH
SUB = CH // 2


def kernel(x, Wq, K_ext, V_ext, Wo):
    my_i = lax.axis_index("i")
    k_loc = lax.dynamic_slice(K_ext, (0, 0, my_i * H_LOC, 0), (B, SKV, H_LOC, DH))
    v_loc = lax.dynamic_slice(V_ext, (0, 0, my_i * H_LOC, 0), (B, SKV, H_LOC, DH))

    def body(x_ref, wq_ref, k_ref, v_ref, wo_ref, out_ref,
             mine_ref, rs_ref, red_ref, ag_ref,
             rs_send, rs_recv, ag_send, ag_recv):
        me = lax.axis_index("i")

        barrier_sem = pltpu.get_barrier_semaphore()
        for d in range(1, N_DEV):
            tgt = lax.rem(me + d, N_DEV)
            pl.semaphore_signal(
                barrier_sem, inc=1,
                device_id=(tgt,), device_id_type=pl.DeviceIdType.MESH,
            )

        x2d = x_ref[...].reshape(ROWS, DMODEL)
        q = jnp.dot(x2d, wq_ref[...],
                    preferred_element_type=jnp.float32) * 0.125

        qb = lax.broadcasted_iota(jnp.int32, (SQ, SKV), 0) // 64
        kb = lax.broadcasted_iota(jnp.int32, (SQ, SKV), 1) // 64
        mask = (qb == kb) | ((kb % 4) == (qb % 4))
        mask_add = jnp.where(mask, 0.0, -1e9).astype(jnp.float32)

        ctx_rows = []
        for b in range(B):
            ctx_cols = []
            for h in range(H_LOC):
                q_bh = q[b * SQ:(b + 1) * SQ, h * DH:(h + 1) * DH]
                k_bh = k_ref[b, :, h, :]
                v_bh = v_ref[b, :, h, :]
                s = lax.dot_general(
                    q_bh, k_bh, (((1,), (1,)), ((), ())),
                    preferred_element_type=jnp.float32,
                ) + mask_add
                w = jnp.exp(s)
                w = w / jnp.sum(w, axis=-1, keepdims=True)
                ctx_cols.append(
                    jnp.dot(w, v_bh, preferred_element_type=jnp.float32))
            ctx_rows.append(jnp.concatenate(ctx_cols, axis=1))
        ctx = jnp.concatenate(ctx_rows, axis=0)
        partial = jnp.dot(ctx, wo_ref[...],
                          preferred_element_type=jnp.float32)
        mine_ref[...] = partial.astype(jnp.bfloat16)

        pl.semaphore_wait(barrier_sem, N_DEV - 1)
        SEND_ORDER = [2, 6, 1, 3, 5, 7, 4]
        RECV_ORDER = [4, 1, 3, 5, 7, 6, 2]

        def rs_desc(d, half):
            tgt = lax.rem(me + d, N_DEV)
            return pltpu.make_async_remote_copy(
                src_ref=mine_ref.at[pl.ds(tgt * CH + half * SUB, SUB), :],
                dst_ref=rs_ref.at[d - 1, pl.ds(half * SUB, SUB), :],
                send_sem=rs_send.at[d - 1, half],
                recv_sem=rs_recv.at[d - 1, half],
                device_id=(tgt,),
                device_id_type=pl.DeviceIdType.MESH,
            )

        def ag_desc(d, half):
            tgt = lax.rem(me + d, N_DEV)
            return pltpu.make_async_remote_copy(
                src_ref=red_ref.at[pl.ds(half * SUB, SUB), :],
                dst_ref=ag_ref.at[d - 1, pl.ds(half * SUB, SUB), :],
                send_sem=ag_send.at[d - 1, half],
                recv_sem=ag_recv.at[d - 1, half],
                device_id=(tgt,),
                device_id_type=pl.DeviceIdType.MESH,
            )

        rsd = {(d, h): rs_desc(d, h) for d in range(1, N_DEV) for h in (0, 1)}
        agd = {(d, h): ag_desc(d, h) for d in range(1, N_DEV) for h in (0, 1)}
        for d in SEND_ORDER:
            rsd[(d, 0)].start()
        for d in SEND_ORDER:
            rsd[(d, 1)].start()

        def out_store(c, half, val_f32):
            out_ref[pl.ds(c // CPB, 1),
                    pl.ds(lax.rem(c, CPB) * CH + half * SUB, SUB), :] = (
                val_f32.reshape(1, SUB, DMODEL))

        red_a = mine_ref[pl.ds(me * CH, SUB), :].astype(jnp.float32)
        for d in RECV_ORDER:
            rsd[(d, 0)].wait_recv()
            red_a = red_a + rs_ref[d - 1, 0:SUB, :].astype(jnp.float32)
        red_ref[pl.ds(0, SUB), :] = red_a.astype(jnp.bfloat16)
        for d in SEND_ORDER:
            agd[(d, 0)].start()
        out_store(me, 0, red_a)

        red_b = mine_ref[pl.ds(me * CH + SUB, SUB), :].astype(jnp.float32)
        for d in RECV_ORDER:
            rsd[(d, 1)].wait_recv()
            red_b = red_b + rs_ref[d - 1, SUB:CH, :].astype(jnp.float32)
        red_ref[pl.ds(SUB, SUB), :] = red_b.astype(jnp.bfloat16)
        for d in SEND_ORDER:
            agd[(d, 1)].start()
        out_store(me, 1, red_b)

        for h in (0, 1):
            for d in RECV_ORDER:
                agd[(d, h)].wait_recv()
                src = lax.rem(me - d + N_DEV, N_DEV)
                out_store(src, h, ag_ref[d - 1, h * SUB:(h + 1) * SUB, :]
                          .astype(jnp.float32))

        for d in range(1, N_DEV):
            for h in (0, 1):
                rsd[(d, h)].wait_send()
                agd[(d, h)].wait_send()

    return pl.pallas_call(
        body,
        out_shape=jax.ShapeDtypeStruct((B, SQ, DMODEL), jnp.float32),
        in_specs=[pl.BlockSpec(memory_space=pltpu.VMEM)] * 5,
        out_specs=pl.BlockSpec(memory_space=pltpu.VMEM),
        scratch_shapes=[
            pltpu.VMEM((ROWS, DMODEL), jnp.bfloat16),
            pltpu.VMEM((N_DEV - 1, CH, DMODEL), jnp.bfloat16),
            pltpu.VMEM((CH, DMODEL), jnp.bfloat16),
            pltpu.VMEM((N_DEV - 1, CH, DMODEL), jnp.bfloat16),
            pltpu.SemaphoreType.DMA((N_DEV - 1, 2)),
            pltpu.SemaphoreType.DMA((N_DEV - 1, 2)),
            pltpu.SemaphoreType.DMA((N_DEV - 1, 2)),
            pltpu.SemaphoreType.DMA((N_DEV - 1, 2)),
        ],
        compiler_params=pltpu.CompilerParams(collective_id=0),
    )(x, Wq, k_loc, v_loc, Wo)
